# baseline (speedup 1.0000x reference)
import functools

import jax
import jax.numpy as jnp
from jax import lax
from jax.experimental import pallas as pl
from jax.experimental.pallas import tpu as pltpu
from jax.experimental.pallas import tpu_sc as plsc

N = 10000
E = 320000
NC = 2
NS = 16
L = 128
NW = NC * NS
GPW = 160
E_PAD = NS * GPW * L
NPAD = 10048
RPT = NPAD // NS
CW = 8
SB = 1
STEPS = GPW // SB
NB = 5
GA = 3


def _sc_aggregate(width, with_cnt):
    out_type = [jax.ShapeDtypeStruct((NPAD, width), jnp.float32)]
    scratch = [
        pltpu.VMEM((GPW, L), jnp.int32),
        pltpu.VMEM((GPW, L), jnp.int32),
        [pltpu.VMEM((L, width), jnp.float32) for _ in range(NB)],
        pltpu.VMEM_SHARED((NPAD, width), jnp.float32),
        [pltpu.SemaphoreType.DMA for _ in range(NB)],
        [pltpu.SemaphoreType.DMA for _ in range(NB)],
    ]
    if with_cnt:
        out_type.append(jax.ShapeDtypeStruct((NPAD, CW), jnp.float32))
        scratch += [
            pltpu.VMEM((L, CW), jnp.float32),
            pltpu.VMEM_SHARED((NPAD, CW), jnp.float32),
        ]

    mesh = plsc.VectorSubcoreMesh(
        core_axis_name="c", subcore_axis_name="s", num_cores=NC, num_subcores=NS
    )

    def body(*refs):
        if with_cnt:
            (p_hbm, srcg, dstg, zeros_w, zeros_c, ones_c,
             agg_out, cnt_out,
             sidx, didx, rows, agg_sh, sem_g, sem_s,
             ones_v, cnt_sh) = refs
        else:
            (p_hbm, srcg, dstg, zeros_w,
             agg_out,
             sidx, didx, rows, agg_sh, sem_g, sem_s) = refs

        c = lax.axis_index("c")
        s = lax.axis_index("s")

        @pl.when(c == 0)
        def _():
            base = s * GPW
            pltpu.sync_copy(srcg.at[pl.ds(base, GPW)], sidx)
            pltpu.sync_copy(dstg.at[pl.ds(base, GPW)], didx)
            pltpu.sync_copy(zeros_w, agg_sh.at[pl.ds(s * RPT, RPT)])
            if with_cnt:
                pltpu.sync_copy(zeros_c, cnt_sh.at[pl.ds(s * RPT, RPT)])
                pltpu.sync_copy(ones_c, ones_v)
            plsc.subcore_barrier()

            def gidx(t):
                return sidx.at[t]

            def scat_idx(t):
                return didx.at[t]

            for k in range(GA):
                pltpu.async_copy(p_hbm.at[gidx(k)], rows[k], sem_g[k])

            @pl.loop(0, GPW // NB)
            def _(i):
                for k in range(NB):
                    t = i * NB + k
                    pltpu.make_async_copy(p_hbm.at[gidx(t)], rows[k],
                                          sem_g[k]).wait()
                    pltpu.async_copy(rows[k], agg_sh.at[scat_idx(t)],
                                     sem_s[k], add=True)
                    if with_cnt:
                        pltpu.async_copy(ones_v, cnt_sh.at[scat_idx(t)],
                                         sem_s[k], add=True)
                    kn = (k + GA) % NB
                    tn = t + GA

                    @pl.when(tn - NB >= 0)
                    def _():
                        pltpu.make_async_copy(rows[kn],
                                              agg_sh.at[scat_idx(t)],
                                              sem_s[kn]).wait()
                        if with_cnt:
                            pltpu.make_async_copy(ones_v,
                                                  cnt_sh.at[scat_idx(t)],
                                                  sem_s[kn]).wait()

                    @pl.when(tn < GPW)
                    def _():
                        pltpu.async_copy(p_hbm.at[gidx(tn)], rows[kn],
                                         sem_g[kn])

            for k in range(NB - (NB - GA), NB):
                pltpu.make_async_copy(rows[k], agg_sh.at[scat_idx(0)],
                                      sem_s[k]).wait()
                if with_cnt:
                    pltpu.make_async_copy(ones_v, cnt_sh.at[scat_idx(0)],
                                          sem_s[k]).wait()

            plsc.subcore_barrier()
            pltpu.sync_copy(agg_sh.at[pl.ds(s * RPT, RPT)],
                            agg_out.at[pl.ds(s * RPT, RPT)])
            if with_cnt:
                pltpu.sync_copy(cnt_sh.at[pl.ds(s * RPT, RPT)],
                                cnt_out.at[pl.ds(s * RPT, RPT)])

    return pl.kernel(
        body,
        out_type=out_type,
        mesh=mesh,
        scratch_types=scratch,
        compiler_params=pltpu.CompilerParams(use_tc_tiling_on_sc=False),
    )


_BLK = 2000


def _tc_prep(x, Wl, Wr, b):
    din, dout = Wl.shape

    def body(x_ref, wl_ref, wr_ref, b_ref, p_ref, r_ref):
        xb = x_ref[...]
        p_ref[...] = jnp.dot(xb, wl_ref[...], preferred_element_type=jnp.float32)
        r_ref[...] = (jnp.dot(xb, wr_ref[...], preferred_element_type=jnp.float32)
                      + b_ref[...])

    return pl.pallas_call(
        body,
        grid=(N // _BLK,),
        in_specs=[
            pl.BlockSpec((_BLK, din), lambda i: (i, 0)),
            pl.BlockSpec((din, dout), lambda i: (0, 0)),
            pl.BlockSpec((din, dout), lambda i: (0, 0)),
            pl.BlockSpec((1, dout), lambda i: (0, 0)),
        ],
        out_specs=[
            pl.BlockSpec((_BLK, dout), lambda i: (i, 0)),
            pl.BlockSpec((_BLK, dout), lambda i: (i, 0)),
        ],
        out_shape=[
            jax.ShapeDtypeStruct((N, dout), jnp.float32),
            jax.ShapeDtypeStruct((N, dout), jnp.float32),
        ],
    )(x, Wl, Wr, b.reshape(1, -1))


def _tc_combine(agg, cnt, r, Wl, Wr, b):
    din, dout = Wl.shape

    def body(aa, ca, r_ref, wl_ref, wr_ref, b_ref, p_ref, rn_ref):
        cnt_col = ca[...][:, 0:1]
        mean = aa[...] / jnp.maximum(cnt_col, 1.0)
        h = jnp.maximum(mean + r_ref[...], 0.0)
        p_ref[...] = jnp.dot(h, wl_ref[...], preferred_element_type=jnp.float32)
        rn_ref[...] = (jnp.dot(h, wr_ref[...], preferred_element_type=jnp.float32)
                       + b_ref[...])

    return pl.pallas_call(
        body,
        grid=(N // _BLK,),
        in_specs=[
            pl.BlockSpec((_BLK, din), lambda i: (i, 0)),
            pl.BlockSpec((_BLK, CW), lambda i: (i, 0)),
            pl.BlockSpec((_BLK, din), lambda i: (i, 0)),
            pl.BlockSpec((din, dout), lambda i: (0, 0)),
            pl.BlockSpec((din, dout), lambda i: (0, 0)),
            pl.BlockSpec((1, dout), lambda i: (0, 0)),
        ],
        out_specs=[
            pl.BlockSpec((_BLK, dout), lambda i: (i, 0)),
            pl.BlockSpec((_BLK, dout), lambda i: (i, 0)),
        ],
        out_shape=[
            jax.ShapeDtypeStruct((N, dout), jnp.float32),
            jax.ShapeDtypeStruct((N, dout), jnp.float32),
        ],
    )(agg, cnt, r, Wl, Wr, b.reshape(1, -1))


def _tc_final(agg, cnt, r):
    def body(aa, ca, r_ref, o_ref):
        cnt_col = ca[...][:, 0:1]
        o_ref[...] = aa[...] / jnp.maximum(cnt_col, 1.0) + r_ref[...]

    spec = pl.BlockSpec((_BLK, CW), lambda i: (i, 0))
    return pl.pallas_call(
        body,
        grid=(N // _BLK,),
        in_specs=[spec] * 3,
        out_specs=spec,
        out_shape=jax.ShapeDtypeStruct((N, CW), jnp.float32),
    )(agg, cnt, r)


def kernel(x, edge_index, Wl0, Wr0, b0, Wl1, Wr1, b1, Wl2, Wr2, b2, Wl3, Wr3, b3):
    src = edge_index[0].astype(jnp.int32)
    dst = edge_index[1].astype(jnp.int32)
    pad = E_PAD - E
    srcg = jnp.concatenate([src, jnp.zeros((pad,), jnp.int32)]).reshape(-1, L)
    fake_dst = N + (jnp.arange(pad, dtype=jnp.int32) % (NPAD - N))
    dstg = jnp.concatenate([dst, fake_dst]).reshape(-1, L)

    zeros64 = jnp.zeros((RPT, 64), jnp.float32)
    zeros_c = jnp.zeros((RPT, CW), jnp.float32)
    ones_c = jnp.ones((L, CW), jnp.float32)

    Wl3p = jnp.pad(Wl3, ((0, 0), (0, CW - Wl3.shape[1])))
    Wr3p = jnp.pad(Wr3, ((0, 0), (0, CW - Wr3.shape[1])))
    b3p = jnp.pad(b3, (0, CW - b3.shape[0]))

    agg64c = _sc_aggregate(64, True)
    agg64 = _sc_aggregate(64, False)
    agg8 = _sc_aggregate(CW, False)

    p, r = _tc_prep(x, Wl0, Wr0, b0)
    agg, cnt = agg64c(p, srcg, dstg, zeros64, zeros_c, ones_c)
    cnt_n = cnt[:N]
    p, r = _tc_combine(agg[:N], cnt_n, r, Wl1, Wr1, b1)
    (agg,) = agg64(p, srcg, dstg, zeros64)
    p, r = _tc_combine(agg[:N], cnt_n, r, Wl2, Wr2, b2)
    (agg,) = agg64(p, srcg, dstg, zeros64)
    p, r = _tc_combine(agg[:N], cnt_n, r, Wl3p, Wr3p, b3p)
    (agg,) = agg8(p, srcg, dstg, zeros_c)
    out = _tc_final(agg[:N], cnt_n, r)
    return out[:, 0]


# speedup vs baseline: 1.1349x; 1.1349x over previous
import functools

import jax
import jax.numpy as jnp
from jax import lax
from jax.experimental import pallas as pl
from jax.experimental.pallas import tpu as pltpu
from jax.experimental.pallas import tpu_sc as plsc

N = 10000
E = 320000
NC = 2
NS = 16
L = 128
NW = NC * NS
GPW = 80
E_PAD = NW * GPW * L
G0 = 145
G1 = 15
GMAX = max(G0, G1)
NPAD = 10048
RPT = NPAD // NS
CW = 8
SB = 1
STEPS = GPW // SB
NB = 5
GA = 4


def _sc_aggregate(width, with_cnt):
    out_type = [jax.ShapeDtypeStruct((NC, NPAD, width), jnp.float32)]
    scratch = [
        pltpu.VMEM((GMAX, SB * L), jnp.int32),
        pltpu.VMEM((GMAX, SB * L), jnp.int32),
        [pltpu.VMEM((SB * L, width), jnp.float32) for _ in range(NB)],
        pltpu.VMEM_SHARED((NPAD, width), jnp.float32),
        [pltpu.SemaphoreType.DMA for _ in range(NB)],
        [pltpu.SemaphoreType.DMA for _ in range(NB)],
    ]
    if with_cnt:
        out_type.append(jax.ShapeDtypeStruct((NC, NPAD, CW), jnp.float32))
        scratch += [
            pltpu.VMEM((SB * L, CW), jnp.float32),
            pltpu.VMEM_SHARED((NPAD, CW), jnp.float32),
        ]

    mesh = plsc.VectorSubcoreMesh(
        core_axis_name="c", subcore_axis_name="s", num_cores=NC, num_subcores=NS
    )

    def body(*refs):
        if with_cnt:
            (p_hbm, srcg, dstg, zeros_w, zeros_c, ones_c,
             agg_out, cnt_out,
             sidx, didx, rows, agg_sh, sem_g, sem_s,
             ones_v, cnt_sh) = refs
        else:
            (p_hbm, srcg, dstg, zeros_w,
             agg_out,
             sidx, didx, rows, agg_sh, sem_g, sem_s) = refs

        c = lax.axis_index("c")
        s = lax.axis_index("s")
        base = jnp.where(c == 0, s * G0, NS * G0 + s * G1)
        msteps = jnp.where(c == 0, G0, G1)

        @pl.when(c == 0)
        def _():
            pltpu.sync_copy(srcg.at[pl.ds(s * G0, G0)], sidx.at[pl.ds(0, G0)])
            pltpu.sync_copy(dstg.at[pl.ds(s * G0, G0)], didx.at[pl.ds(0, G0)])

        @pl.when(c == 1)
        def _():
            b1 = NS * G0 + s * G1
            pltpu.sync_copy(srcg.at[pl.ds(b1, G1)], sidx.at[pl.ds(0, G1)])
            pltpu.sync_copy(dstg.at[pl.ds(b1, G1)], didx.at[pl.ds(0, G1)])

        pltpu.sync_copy(zeros_w, agg_sh.at[pl.ds(s * RPT, RPT)])
        if with_cnt:
            pltpu.sync_copy(ones_c, ones_v)
            pltpu.sync_copy(zeros_c, cnt_sh.at[pl.ds(s * RPT, RPT)])
        plsc.subcore_barrier()

        def gidx(t):
            return sidx.at[t]

        def scat_idx(t):
            return didx.at[t]

        for k in range(GA):
            pltpu.async_copy(p_hbm.at[gidx(k)], rows[k], sem_g[k])

        @pl.loop(0, msteps // NB)
        def _(i):
            for k in range(NB):
                t = i * NB + k
                pltpu.make_async_copy(p_hbm.at[gidx(t)], rows[k],
                                      sem_g[k]).wait()
                pltpu.async_copy(rows[k], agg_sh.at[scat_idx(t)], sem_s[k],
                                 add=True)
                if with_cnt:
                    pltpu.async_copy(ones_v, cnt_sh.at[scat_idx(t)], sem_s[k],
                                     add=True)
                kn = (k + GA) % NB
                tn = t + GA

                @pl.when(tn - NB >= 0)
                def _():
                    pltpu.make_async_copy(rows[kn], agg_sh.at[scat_idx(t)],
                                          sem_s[kn]).wait()
                    if with_cnt:
                        pltpu.make_async_copy(ones_v, cnt_sh.at[scat_idx(t)],
                                              sem_s[kn]).wait()

                @pl.when(tn < msteps)
                def _():
                    pltpu.async_copy(p_hbm.at[gidx(tn)], rows[kn], sem_g[kn])

        for t in range(STEPS - (NB - GA), STEPS):
            k = t % NB
            pltpu.make_async_copy(rows[k], agg_sh.at[scat_idx(0)],
                                  sem_s[k]).wait()
            if with_cnt:
                pltpu.make_async_copy(ones_v, cnt_sh.at[scat_idx(0)],
                                      sem_s[k]).wait()

        plsc.subcore_barrier()
        pltpu.sync_copy(agg_sh.at[pl.ds(s * RPT, RPT)],
                        agg_out.at[c, pl.ds(s * RPT, RPT)])
        if with_cnt:
            pltpu.sync_copy(cnt_sh.at[pl.ds(s * RPT, RPT)],
                            cnt_out.at[c, pl.ds(s * RPT, RPT)])

    return pl.kernel(
        body,
        out_type=out_type,
        mesh=mesh,
        scratch_types=scratch,
        compiler_params=pltpu.CompilerParams(use_tc_tiling_on_sc=False),
    )


_BLK = 2000


def _tc_prep(x, Wl, Wr, b):
    din, dout = Wl.shape

    def body(x_ref, wl_ref, wr_ref, b_ref, p_ref, r_ref):
        xb = x_ref[...]
        p_ref[...] = jnp.dot(xb, wl_ref[...], preferred_element_type=jnp.float32)
        r_ref[...] = (jnp.dot(xb, wr_ref[...], preferred_element_type=jnp.float32)
                      + b_ref[...])

    return pl.pallas_call(
        body,
        grid=(N // _BLK,),
        in_specs=[
            pl.BlockSpec((_BLK, din), lambda i: (i, 0)),
            pl.BlockSpec((din, dout), lambda i: (0, 0)),
            pl.BlockSpec((din, dout), lambda i: (0, 0)),
            pl.BlockSpec((1, dout), lambda i: (0, 0)),
        ],
        out_specs=[
            pl.BlockSpec((_BLK, dout), lambda i: (i, 0)),
            pl.BlockSpec((_BLK, dout), lambda i: (i, 0)),
        ],
        out_shape=[
            jax.ShapeDtypeStruct((N, dout), jnp.float32),
            jax.ShapeDtypeStruct((N, dout), jnp.float32),
        ],
    )(x, Wl, Wr, b.reshape(1, -1))


def _tc_combine(agg_a, agg_b, cnt_a, cnt_b, r, Wl, Wr, b):
    din, dout = Wl.shape

    def body(aa, ab, ca, cb, r_ref, wl_ref, wr_ref, b_ref, p_ref, rn_ref):
        cnt = ca[...][:, 0:1] + cb[...][:, 0:1]
        mean = (aa[...] + ab[...]) / jnp.maximum(cnt, 1.0)
        h = jnp.maximum(mean + r_ref[...], 0.0)
        p_ref[...] = jnp.dot(h, wl_ref[...], preferred_element_type=jnp.float32)
        rn_ref[...] = (jnp.dot(h, wr_ref[...], preferred_element_type=jnp.float32)
                       + b_ref[...])

    return pl.pallas_call(
        body,
        grid=(N // _BLK,),
        in_specs=[
            pl.BlockSpec((_BLK, din), lambda i: (i, 0)),
            pl.BlockSpec((_BLK, din), lambda i: (i, 0)),
            pl.BlockSpec((_BLK, CW), lambda i: (i, 0)),
            pl.BlockSpec((_BLK, CW), lambda i: (i, 0)),
            pl.BlockSpec((_BLK, din), lambda i: (i, 0)),
            pl.BlockSpec((din, dout), lambda i: (0, 0)),
            pl.BlockSpec((din, dout), lambda i: (0, 0)),
            pl.BlockSpec((1, dout), lambda i: (0, 0)),
        ],
        out_specs=[
            pl.BlockSpec((_BLK, dout), lambda i: (i, 0)),
            pl.BlockSpec((_BLK, dout), lambda i: (i, 0)),
        ],
        out_shape=[
            jax.ShapeDtypeStruct((N, dout), jnp.float32),
            jax.ShapeDtypeStruct((N, dout), jnp.float32),
        ],
    )(agg_a, agg_b, cnt_a, cnt_b, r, Wl, Wr, b.reshape(1, -1))


def _tc_final(agg_a, agg_b, cnt_a, cnt_b, r):
    def body(aa, ab, ca, cb, r_ref, o_ref):
        cnt = ca[...][:, 0:1] + cb[...][:, 0:1]
        o_ref[...] = (aa[...] + ab[...]) / jnp.maximum(cnt, 1.0) + r_ref[...]

    spec = pl.BlockSpec((_BLK, CW), lambda i: (i, 0))
    return pl.pallas_call(
        body,
        grid=(N // _BLK,),
        in_specs=[spec] * 5,
        out_specs=spec,
        out_shape=jax.ShapeDtypeStruct((N, CW), jnp.float32),
    )(agg_a, agg_b, cnt_a, cnt_b, r)


def kernel(x, edge_index, Wl0, Wr0, b0, Wl1, Wr1, b1, Wl2, Wr2, b2, Wl3, Wr3, b3):
    src = edge_index[0].astype(jnp.int32)
    dst = edge_index[1].astype(jnp.int32)
    pad = E_PAD - E
    srcg = jnp.concatenate(
        [src, jnp.zeros((pad + GMAX * SB * L,), jnp.int32)]).reshape(-1, SB * L)
    fake_dst = N + (jnp.arange(pad + GMAX * SB * L, dtype=jnp.int32) % (NPAD - N))
    dstg = jnp.concatenate([dst, fake_dst]).reshape(-1, SB * L)

    ones_c = jnp.ones((SB * L, CW), jnp.float32)
    zeros_c = jnp.zeros((RPT, CW), jnp.float32)
    zeros64 = jnp.zeros((RPT, 64), jnp.float32)

    Wl3p = jnp.pad(Wl3, ((0, 0), (0, CW - Wl3.shape[1])))
    Wr3p = jnp.pad(Wr3, ((0, 0), (0, CW - Wr3.shape[1])))
    b3p = jnp.pad(b3, (0, CW - b3.shape[0]))

    agg64c = _sc_aggregate(64, True)
    agg64 = _sc_aggregate(64, False)
    agg8 = _sc_aggregate(CW, False)

    p, r = _tc_prep(x, Wl0, Wr0, b0)
    agg, cnt = agg64c(p, srcg, dstg, zeros64, zeros_c, ones_c)
    cnt_a, cnt_b = cnt[0, :N], cnt[1, :N]
    p, r = _tc_combine(agg[0, :N], agg[1, :N], cnt_a, cnt_b, r, Wl1, Wr1, b1)
    (agg,) = agg64(p, srcg, dstg, zeros64)
    p, r = _tc_combine(agg[0, :N], agg[1, :N], cnt_a, cnt_b, r, Wl2, Wr2, b2)
    (agg,) = agg64(p, srcg, dstg, zeros64)
    p, r = _tc_combine(agg[0, :N], agg[1, :N], cnt_a, cnt_b, r, Wl3p, Wr3p, b3p)
    (agg,) = agg8(p, srcg, dstg, zeros_c)
    out = _tc_final(agg[0, :N], agg[1, :N], cnt_a, cnt_b, r)
    return out[:, 0]
